# baseline (speedup 1.0000x reference)
import functools

import jax
import jax.numpy as jnp
from jax import lax
from jax.experimental import pallas as pl
from jax.experimental.pallas import tpu as pltpu
from jax.experimental.pallas import tpu_sc as plsc

B = 16384
K = 200
R = 1000
D = 16
NC, NS = 2, 16
NW = NC * NS
BPW = B // NW
CB = 512


@functools.cache
def _make_sc_gather():
    @functools.partial(
        pl.kernel,
        mesh=plsc.VectorSubcoreMesh(core_axis_name="c", subcore_axis_name="s"),
        out_type=jax.ShapeDtypeStruct((B, D), jnp.float32),
        scratch_types=[
            pltpu.VMEM((BPW,), jnp.int32),
            pltpu.VMEM((BPW, D), jnp.float32),
            pltpu.SemaphoreType.DMA,
        ],
        compiler_params=pltpu.CompilerParams(use_tc_tiling_on_sc=False),
    )
    def _sc_gather(table_hbm, idx_hbm, out_hbm, idx_v, rows_v, sem):
        wid = lax.axis_index("s") * NC + lax.axis_index("c")
        base = wid * BPW
        pltpu.sync_copy(idx_hbm.at[pl.ds(base, BPW)], idx_v)
        pltpu.async_copy(table_hbm.at[idx_v], rows_v, sem).wait()
        pltpu.sync_copy(rows_v, out_hbm.at[pl.ds(base, BPW)])

    return _sc_gather


def _softplus(x):
    return jnp.maximum(x, 0.0) + jnp.log1p(jnp.exp(-jnp.abs(x)))


def _tc_body(p_ref, base_ref, ssr_ref, dsr_ref, fsr_ref, sso_ref, dso_ref,
             fso_ref, sro_ref, dro_ref, fro_ref, adj_ref, bias_ref):
    p = p_ref[...]

    def row(j):
        return p[j:j + 1, :]

    def branch(seen, dt, freq, lam, wrec, wfreq, bias, wstale=None):
        dt_feat = jnp.log1p(jnp.maximum(dt, 0.0))
        rec = jnp.exp(-lam * dt_feat) * seen
        ff = jnp.log1p(jnp.maximum(freq, 0.0))
        denom = jnp.maximum(jnp.max(ff, axis=0, keepdims=True), 1e-8)
        score = wrec * rec + (wfreq / denom) * ff * seen + bias
        if wstale is not None:
            score = score - wstale * (1.0 - rec) * seen
        return jnp.tanh(score) * seen

    g_sr = branch(ssr_ref[...], dsr_ref[...], fsr_ref[...],
                  _softplus(row(0)), row(1), row(2), row(4), wstale=row(3))
    g_so = branch(sso_ref[...], dso_ref[...], fso_ref[...],
                  _softplus(row(5)), row(6), row(7), row(8))
    g_ro = branch(sro_ref[...], dro_ref[...], fro_ref[...],
                  _softplus(row(9)), row(10), row(11), row(12))

    hist = 0.005 * g_sr + 0.04 * (g_so + g_ro)
    adj_ref[...] = base_ref[...] + hist
    bias_ref[...] = hist


_col_spec = pl.BlockSpec((K, CB), lambda i: (0, i))
_par_spec = pl.BlockSpec((D, CB), lambda i: (0, i))

_tc_compute = pl.pallas_call(
    _tc_body,
    grid=(B // CB,),
    in_specs=[_par_spec] + [_col_spec] * 10,
    out_specs=[_col_spec, _col_spec],
    out_shape=[jax.ShapeDtypeStruct((K, B), jnp.float32)] * 2,
)


def kernel(base_scores, rel_ids, seen_sr, dt_sr, freq_sr, seen_so, dt_so,
           freq_so, seen_ro, dt_ro, freq_ro, params):
    tab = jnp.concatenate([
        params['lam_sr'], params['wrec_sr'], params['wfreq_sr'],
        params['wstale_sr'], params['bias_sr'],
        params['lam_so'], params['wrec_so'], params['wfreq_so'],
        params['bias_so'],
        params['lam_ro'], params['wrec_ro'], params['wfreq_ro'],
        params['bias_ro'],
        jnp.zeros((R, 3), jnp.float32),
    ], axis=1)
    gathered = _make_sc_gather()(tab, rel_ids.astype(jnp.int32))
    adj_t, hb_t = _tc_compute(
        gathered.T, base_scores.T, seen_sr.T, dt_sr.T, freq_sr.T,
        seen_so.T, dt_so.T, freq_so.T, seen_ro.T, dt_ro.T, freq_ro.T)
    return (adj_t.T, hb_t.T)


# speedup vs baseline: 1.1039x; 1.1039x over previous
import functools

import jax
import jax.numpy as jnp
from jax import lax
from jax.experimental import pallas as pl
from jax.experimental.pallas import tpu as pltpu
from jax.experimental.pallas import tpu_sc as plsc

B = 16384
K = 200
R = 1000
D = 16
NC, NS = 2, 16
NW = NC * NS
BPW = B // NW
CB = 2048


@functools.cache
def _make_sc_gather():
    @functools.partial(
        pl.kernel,
        mesh=plsc.VectorSubcoreMesh(core_axis_name="c", subcore_axis_name="s"),
        out_type=jax.ShapeDtypeStruct((B, D), jnp.float32),
        scratch_types=[
            pltpu.VMEM((BPW,), jnp.int32),
            pltpu.VMEM((BPW, D), jnp.float32),
            pltpu.SemaphoreType.DMA,
        ],
        compiler_params=pltpu.CompilerParams(use_tc_tiling_on_sc=False),
    )
    def _sc_gather(table_hbm, idx_hbm, out_hbm, idx_v, rows_v, sem):
        wid = lax.axis_index("s") * NC + lax.axis_index("c")
        base = wid * BPW
        pltpu.sync_copy(idx_hbm.at[pl.ds(base, BPW)], idx_v)
        pltpu.async_copy(table_hbm.at[idx_v], rows_v, sem).wait()
        pltpu.sync_copy(rows_v, out_hbm.at[pl.ds(base, BPW)])

    return _sc_gather


def _softplus(x):
    return jnp.maximum(x, 0.0) + jnp.log1p(jnp.exp(-jnp.abs(x)))


def _tc_body(p_ref, base_ref, ssr_ref, dsr_ref, fsr_ref, sso_ref, dso_ref,
             fso_ref, sro_ref, dro_ref, fro_ref, adj_ref, bias_ref):
    p = p_ref[...]

    def row(j):
        return p[j:j + 1, :]

    def branch(seen, dt, freq, lam, wrec, wfreq, bias, wstale=None):
        dt_feat = jnp.log1p(jnp.maximum(dt, 0.0))
        rec = jnp.exp(-lam * dt_feat) * seen
        ff = jnp.log1p(jnp.maximum(freq, 0.0))
        denom = jnp.maximum(jnp.max(ff, axis=0, keepdims=True), 1e-8)
        score = wrec * rec + (wfreq / denom) * ff * seen + bias
        if wstale is not None:
            score = score - wstale * (1.0 - rec) * seen
        return jnp.tanh(score) * seen

    g_sr = branch(ssr_ref[...], dsr_ref[...], fsr_ref[...],
                  _softplus(row(0)), row(1), row(2), row(4), wstale=row(3))
    g_so = branch(sso_ref[...], dso_ref[...], fso_ref[...],
                  _softplus(row(5)), row(6), row(7), row(8))
    g_ro = branch(sro_ref[...], dro_ref[...], fro_ref[...],
                  _softplus(row(9)), row(10), row(11), row(12))

    hist = 0.005 * g_sr + 0.04 * (g_so + g_ro)
    adj_ref[...] = base_ref[...] + hist
    bias_ref[...] = hist


_col_spec = pl.BlockSpec((K, CB), lambda i: (0, i))
_par_spec = pl.BlockSpec((D, CB), lambda i: (0, i))

_tc_compute = pl.pallas_call(
    _tc_body,
    grid=(B // CB,),
    in_specs=[_par_spec] + [_col_spec] * 10,
    out_specs=[_col_spec, _col_spec],
    out_shape=[jax.ShapeDtypeStruct((K, B), jnp.float32)] * 2,
)


def kernel(base_scores, rel_ids, seen_sr, dt_sr, freq_sr, seen_so, dt_so,
           freq_so, seen_ro, dt_ro, freq_ro, params):
    tab = jnp.concatenate([
        params['lam_sr'], params['wrec_sr'], params['wfreq_sr'],
        params['wstale_sr'], params['bias_sr'],
        params['lam_so'], params['wrec_so'], params['wfreq_so'],
        params['bias_so'],
        params['lam_ro'], params['wrec_ro'], params['wfreq_ro'],
        params['bias_ro'],
        jnp.zeros((R, 3), jnp.float32),
    ], axis=1)
    gathered = _make_sc_gather()(tab, rel_ids.astype(jnp.int32))
    adj_t, hb_t = _tc_compute(
        gathered.T, base_scores.T, seen_sr.T, dt_sr.T, freq_sr.T,
        seen_so.T, dt_so.T, freq_so.T, seen_ro.T, dt_ro.T, freq_ro.T)
    return (adj_t.T, hb_t.T)


# speedup vs baseline: 1.1794x; 1.0684x over previous
import functools

import jax
import jax.numpy as jnp
from jax import lax
from jax.experimental import pallas as pl
from jax.experimental.pallas import tpu as pltpu
from jax.experimental.pallas import tpu_sc as plsc

B = 16384
K = 200
R = 1000
D = 16
NC, NS = 2, 16
NW = NC * NS
BPW = B // NW
CB = 2048


@functools.cache
def _make_sc_gather():
    @functools.partial(
        pl.kernel,
        mesh=plsc.VectorSubcoreMesh(core_axis_name="c", subcore_axis_name="s"),
        out_type=jax.ShapeDtypeStruct((D // 8, B // 128, 8, 128),
                                      jnp.float32),
        scratch_types=[
            pltpu.VMEM((BPW,), jnp.int32),
            pltpu.VMEM((BPW, D), jnp.float32),
            pltpu.VMEM((D // 8, BPW // 128, 8, 128), jnp.float32),
            pltpu.SemaphoreType.DMA,
        ],
        compiler_params=pltpu.CompilerParams(use_tc_tiling_on_sc=False,
                                             needs_layout_passes=False),
    )
    def _sc_gather(table_hbm, idx_hbm, out_hbm, idx_v, rows_v, cols_v, sem):
        wid = lax.axis_index("s") * NC + lax.axis_index("c")
        base = wid * BPW
        nt = BPW // 128
        pltpu.sync_copy(idx_hbm.at[pl.ds(base, BPW)], idx_v)
        pltpu.async_copy(table_hbm.at[idx_v], rows_v, sem).wait()
        lane16 = lax.iota(jnp.int32, 16)
        for j in range(13):
            s, r = divmod(j, 8)
            jv = jnp.full((16,), j, jnp.int32)
            for t in range(nt):
                for l0 in range(0, 128, 16):
                    rv = lane16 + (128 * t + l0)
                    vals = plsc.load_gather(rows_v, [rv, jv])
                    cols_v[s, t, r, l0:l0 + 16] = vals

        pltpu.sync_copy(cols_v, out_hbm.at[:, pl.ds(wid * nt, nt)])

    return _sc_gather


def _softplus(x):
    return jnp.maximum(x, 0.0) + jnp.log1p(jnp.exp(-jnp.abs(x)))


def _tc_body(p_ref, base_ref, ssr_ref, dsr_ref, fsr_ref, sso_ref, dso_ref,
             fso_ref, sro_ref, dro_ref, fro_ref, adj_ref, bias_ref):
    p4 = p_ref[...]

    def row(j):
        s, r = divmod(j, 8)
        return jnp.reshape(p4[s:s + 1, :, r:r + 1, :], (1, CB))

    def branch(seen, dt, freq, lam, wrec, wfreq, bias, wstale=None):
        dt_feat = jnp.log1p(jnp.maximum(dt, 0.0))
        rec = jnp.exp(-lam * dt_feat) * seen
        ff = jnp.log1p(jnp.maximum(freq, 0.0))
        denom = jnp.maximum(jnp.max(ff, axis=0, keepdims=True), 1e-8)
        score = wrec * rec + (wfreq / denom) * ff * seen + bias
        if wstale is not None:
            score = score - wstale * (1.0 - rec) * seen
        return jnp.tanh(score) * seen

    g_sr = branch(ssr_ref[...], dsr_ref[...], fsr_ref[...],
                  _softplus(row(0)), row(1), row(2), row(4), wstale=row(3))
    g_so = branch(sso_ref[...], dso_ref[...], fso_ref[...],
                  _softplus(row(5)), row(6), row(7), row(8))
    g_ro = branch(sro_ref[...], dro_ref[...], fro_ref[...],
                  _softplus(row(9)), row(10), row(11), row(12))

    hist = 0.005 * g_sr + 0.04 * (g_so + g_ro)
    adj_ref[...] = base_ref[...] + hist
    bias_ref[...] = hist


_col_spec = pl.BlockSpec((K, CB), lambda i: (0, i))
_par_spec = pl.BlockSpec((D // 8, CB // 128, 8, 128),
                         lambda i: (0, i, 0, 0))

_tc_compute = pl.pallas_call(
    _tc_body,
    grid=(B // CB,),
    in_specs=[_par_spec] + [_col_spec] * 10,
    out_specs=[_col_spec, _col_spec],
    out_shape=[jax.ShapeDtypeStruct((K, B), jnp.float32)] * 2,
)


def kernel(base_scores, rel_ids, seen_sr, dt_sr, freq_sr, seen_so, dt_so,
           freq_so, seen_ro, dt_ro, freq_ro, params):
    tab = jnp.concatenate([
        params['lam_sr'], params['wrec_sr'], params['wfreq_sr'],
        params['wstale_sr'], params['bias_sr'],
        params['lam_so'], params['wrec_so'], params['wfreq_so'],
        params['bias_so'],
        params['lam_ro'], params['wrec_ro'], params['wfreq_ro'],
        params['bias_ro'],
        jnp.zeros((R, 3), jnp.float32),
    ], axis=1)
    gathered4 = _make_sc_gather()(tab, rel_ids.astype(jnp.int32))
    adj_t, hb_t = _tc_compute(
        gathered4, base_scores.T, seen_sr.T, dt_sr.T, freq_sr.T,
        seen_so.T, dt_so.T, freq_so.T, seen_ro.T, dt_ro.T, freq_ro.T)
    return (adj_t.T, hb_t.T)


# speedup vs baseline: 1.2168x; 1.0317x over previous
import functools

import jax
import jax.numpy as jnp
from jax import lax
from jax.experimental import pallas as pl
from jax.experimental.pallas import tpu as pltpu
from jax.experimental.pallas import tpu_sc as plsc

B = 16384
K = 200
R = 1000
D = 16
NC, NS = 2, 16
NW = NC * NS
BPW = B // NW
CB = 2048


@functools.cache
def _make_sc_gather():
    @functools.partial(
        pl.kernel,
        mesh=plsc.VectorSubcoreMesh(core_axis_name="c", subcore_axis_name="s"),
        out_type=jax.ShapeDtypeStruct((D // 8, B // 128, 8, 128),
                                      jnp.float32),
        scratch_types=[
            pltpu.VMEM((BPW,), jnp.int32),
            pltpu.VMEM((BPW, D), jnp.float32),
            pltpu.VMEM((D // 8, BPW // 128, 8, 128), jnp.float32),
            pltpu.SemaphoreType.DMA,
        ],
        compiler_params=pltpu.CompilerParams(use_tc_tiling_on_sc=False,
                                             needs_layout_passes=False),
    )
    def _sc_gather(table_hbm, idx_hbm, out_hbm, idx_v, rows_v, cols_v, sem):
        wid = lax.axis_index("s") * NC + lax.axis_index("c")
        base = wid * BPW
        nt = BPW // 128
        pltpu.sync_copy(idx_hbm.at[pl.ds(base, BPW)], idx_v)
        pltpu.async_copy(table_hbm.at[idx_v], rows_v, sem).wait()
        lane16 = lax.iota(jnp.int32, 16)
        jvs = [jnp.full((16,), j, jnp.int32) for j in range(13)]

        def chunk(i, carry):
            c0 = i * 16
            t = c0 // 128
            l0 = c0 - t * 128
            rv = lane16 + c0
            for j in range(13):
                s, r = divmod(j, 8)
                vals = plsc.load_gather(rows_v, [rv, jvs[j]])
                cols_v[s, t, r, pl.ds(l0, 16)] = vals
            return carry

        lax.fori_loop(0, BPW // 16, chunk, 0)
        pltpu.sync_copy(cols_v, out_hbm.at[:, pl.ds(wid * nt, nt)])

    return _sc_gather


def _softplus(x):
    return jnp.maximum(x, 0.0) + jnp.log1p(jnp.exp(-jnp.abs(x)))


def _tc_body(p_ref, base_ref, ssr_ref, dsr_ref, fsr_ref, sso_ref, dso_ref,
             fso_ref, sro_ref, dro_ref, fro_ref, adj_ref, bias_ref):
    p4 = p_ref[...]

    def row(j):
        s, r = divmod(j, 8)
        return jnp.reshape(p4[s:s + 1, :, r:r + 1, :], (1, CB))

    def branch(seen, dt, freq, lam, wrec, wfreq, bias, wstale=None):
        dt_feat = jnp.log1p(jnp.maximum(dt, 0.0))
        rec = jnp.exp(-lam * dt_feat) * seen
        ff = jnp.log1p(jnp.maximum(freq, 0.0))
        denom = jnp.maximum(jnp.max(ff, axis=0, keepdims=True), 1e-8)
        score = wrec * rec + (wfreq / denom) * ff * seen + bias
        if wstale is not None:
            score = score - wstale * (1.0 - rec) * seen
        return jnp.tanh(score) * seen

    g_sr = branch(ssr_ref[...], dsr_ref[...], fsr_ref[...],
                  _softplus(row(0)), row(1), row(2), row(4), wstale=row(3))
    g_so = branch(sso_ref[...], dso_ref[...], fso_ref[...],
                  _softplus(row(5)), row(6), row(7), row(8))
    g_ro = branch(sro_ref[...], dro_ref[...], fro_ref[...],
                  _softplus(row(9)), row(10), row(11), row(12))

    hist = 0.005 * g_sr + 0.04 * (g_so + g_ro)
    adj_ref[...] = base_ref[...] + hist
    bias_ref[...] = hist


_col_spec = pl.BlockSpec((K, CB), lambda i: (0, i))
_par_spec = pl.BlockSpec((D // 8, CB // 128, 8, 128),
                         lambda i: (0, i, 0, 0))

_tc_compute = pl.pallas_call(
    _tc_body,
    grid=(B // CB,),
    in_specs=[_par_spec] + [_col_spec] * 10,
    out_specs=[_col_spec, _col_spec],
    out_shape=[jax.ShapeDtypeStruct((K, B), jnp.float32)] * 2,
)


def kernel(base_scores, rel_ids, seen_sr, dt_sr, freq_sr, seen_so, dt_so,
           freq_so, seen_ro, dt_ro, freq_ro, params):
    tab = jnp.concatenate([
        params['lam_sr'], params['wrec_sr'], params['wfreq_sr'],
        params['wstale_sr'], params['bias_sr'],
        params['lam_so'], params['wrec_so'], params['wfreq_so'],
        params['bias_so'],
        params['lam_ro'], params['wrec_ro'], params['wfreq_ro'],
        params['bias_ro'],
        jnp.zeros((R, 3), jnp.float32),
    ], axis=1)
    gathered4 = _make_sc_gather()(tab, rel_ids.astype(jnp.int32))
    adj_t, hb_t = _tc_compute(
        gathered4, base_scores.T, seen_sr.T, dt_sr.T, freq_sr.T,
        seen_so.T, dt_so.T, freq_so.T, seen_ro.T, dt_ro.T, freq_ro.T)
    return (adj_t.T, hb_t.T)
